# baseline (speedup 1.0000x reference)
import jax
import jax.numpy as jnp
from jax import lax
from jax.experimental import pallas as pl
from jax.experimental.pallas import tpu as pltpu
from jax.experimental.pallas import tpu_sc as plsc

N_USER = 25000
N_ITEM = 25000
N = N_USER + N_ITEM
D = 64
H = 32
E = 800000
N_LAYERS = 3

NSUB = 16
GROUP = 128
GPS = 392
NR = GPS // 4
ROWS = GPS * NSUB
EPAD = ROWS * GROUP
NP = 50176
TPN = NP // NSUB
CH = 112
ZCH = 224


def _body(x, epk, acc, cur_a, cur_b, shared, idxb, rows,
          g0, g1, g2, g3, s0, s1, s2, s3, isem):
    c = lax.axis_index("c")
    s = lax.axis_index("s")
    node0 = s * TPN
    grow0 = s * GPS
    gsem = (g0, g1, g2, g3)
    ssem = (s0, s1, s2, s3)

    for layer in range(N_LAYERS):
        tbl = (x, cur_a, cur_b)[layer]
        nxt = (cur_a, cur_b, None)[layer]
        accsrc = x if layer == 0 else acc

        @plsc.parallel_loop(0, ZCH, unroll=4)
        def _(i):
            rows[i, pl.ds(0, 16)] = jnp.zeros((16,), jnp.float32)
            rows[i, pl.ds(16, 16)] = jnp.zeros((16,), jnp.float32)
        zdescs = [
            pltpu.async_copy(rows.at[pl.ds(0, ZCH)],
                             shared.at[pl.ds(node0 + q * ZCH, ZCH)], g0)
            for q in range(TPN // ZCH)
        ]
        for dsc in zdescs:
            dsc.wait()
        plsc.subcore_barrier()

        def fetch(k, buf, wait=False):
            mk = pltpu.make_async_copy if wait else pltpu.async_copy
            dsc = mk(epk.at[pl.ds(grow0 + k * 4, 4)], idxb.at[buf], isem)
            if wait:
                dsc.wait()
            return dsc

        def gth(buf, slot, wait=False):
            mk = pltpu.make_async_copy if wait else pltpu.async_copy
            dsc = mk(tbl.at[c].at[idxb.at[buf, slot, 0]],
                     rows.at[pl.ds(slot * GROUP, GROUP)], gsem[slot])
            if wait:
                dsc.wait()
            return dsc

        def sct(buf, slot, wait=False):
            if wait:
                pltpu.make_async_copy(
                    rows.at[pl.ds(slot * GROUP, GROUP)],
                    shared.at[idxb.at[buf, slot, 1]], ssem[slot]).wait()
            else:
                pltpu.async_copy(rows.at[pl.ds(slot * GROUP, GROUP)],
                                 shared.at[idxb.at[buf, slot, 1]],
                                 ssem[slot], add=True)

        def scale(buf, slot):
            @plsc.parallel_loop(0, 8, unroll=2)
            def _(m):
                w16 = plsc.bitcast(idxb[buf, slot, 2, pl.ds(m * 16, 16)],
                                   jnp.float32)
                for e in range(16):
                    j = slot * GROUP + m * 16 + e
                    wv = jnp.broadcast_to(w16[e], (16,))
                    rows[j, pl.ds(0, 16)] = rows[j, pl.ds(0, 16)] * wv
                    rows[j, pl.ds(16, 16)] = rows[j, pl.ds(16, 16)] * wv

        fetch(0, 0).wait()
        gth(0, 0)
        gth(0, 1)
        fetch(1, 1)

        def round_body(r, _):
            b0 = lax.rem(r, 4)
            b1 = lax.rem(r + 1, 4)
            b2 = lax.rem(r + 2, 4)
            bm = lax.rem(r + 3, 4)

            @pl.when(r < NR - 1)
            def _():
                fetch(r + 1, b1, wait=True)

            @pl.when(r > 0)
            def _():
                sct(bm, 2, wait=True)

            gth(b0, 2)
            gth(b0, 0, wait=True)
            scale(b0, 0)
            sct(b0, 0)

            @pl.when(r > 0)
            def _():
                sct(bm, 3, wait=True)

            gth(b0, 3)

            @pl.when(r < NR - 2)
            def _():
                fetch(r + 2, b2)

            gth(b0, 1, wait=True)
            scale(b0, 1)
            sct(b0, 1)

            gth(b0, 2, wait=True)
            scale(b0, 2)
            sct(b0, 2)
            sct(b0, 0, wait=True)

            @pl.when(r < NR - 1)
            def _():
                gth(b1, 0)

            gth(b0, 3, wait=True)
            scale(b0, 3)
            sct(b0, 3)
            sct(b0, 1, wait=True)

            @pl.when(r < NR - 1)
            def _():
                gth(b1, 1)

            return 0

        lax.fori_loop(0, NR, round_body, 0)
        sct((NR - 1) % 4, 2, wait=True)
        sct((NR - 1) % 4, 3, wait=True)
        plsc.subcore_barrier()

        outb = [rows.at[pl.ds(0, CH)], rows.at[pl.ds(CH, CH)]]
        accb = [rows.at[pl.ds(2 * CH, CH)], rows.at[pl.ds(3 * CH, CH)]]

        def rd(q, p):
            nb = node0 + q * CH
            return (pltpu.async_copy(shared.at[pl.ds(nb, CH)],
                                     outb[p], gsem[p]),
                    pltpu.async_copy(accsrc.at[c].at[pl.ds(nb, CH)],
                                     accb[p], gsem[2 + p]))

        nchunk = TPN // CH
        rdesc = {0: rd(0, 0)}
        wdesc = {}
        for q in range(nchunk):
            p = q % 2
            if q + 1 < nchunk:
                if q - 1 in wdesc:
                    for dsc in wdesc.pop(q - 1):
                        dsc.wait()
                rdesc[q + 1] = rd(q + 1, 1 - p)
            for dsc in rdesc.pop(q):
                dsc.wait()

            o_s, a_s = outb[p], accb[p]

            @plsc.parallel_loop(0, CH, unroll=4)
            def _(i):
                a_s[i, pl.ds(0, 16)] = (a_s[i, pl.ds(0, 16)]
                                        + o_s[i, pl.ds(0, 16)])
                a_s[i, pl.ds(16, 16)] = (a_s[i, pl.ds(16, 16)]
                                         + o_s[i, pl.ds(16, 16)])

            nb = node0 + q * CH
            wd = [pltpu.async_copy(a_s, acc.at[c].at[pl.ds(nb, CH)],
                                   ssem[2 + p])]
            if nxt is not None:
                wd.append(pltpu.async_copy(o_s, nxt.at[c].at[pl.ds(nb, CH)],
                                           ssem[p]))
            wdesc[q] = wd
        for q in list(wdesc):
            for dsc in wdesc.pop(q):
                dsc.wait()
        plsc.subcore_barrier()


@jax.jit
def _propagate(xt, epk):
    mesh = plsc.VectorSubcoreMesh(core_axis_name="c", subcore_axis_name="s")
    f = pl.kernel(
        _body,
        out_type=(
            jax.ShapeDtypeStruct((2, NP, H), jnp.float32),
            jax.ShapeDtypeStruct((2, NP, H), jnp.float32),
            jax.ShapeDtypeStruct((2, NP, H), jnp.float32),
        ),
        mesh=mesh,
        compiler_params=pltpu.CompilerParams(use_tc_tiling_on_sc=False,
                                             needs_layout_passes=False),
        scratch_types=[
            pltpu.VMEM_SHARED((NP, H), jnp.float32),
            pltpu.VMEM((4, 4, 3, GROUP), jnp.int32),
            pltpu.VMEM((4 * GROUP, H), jnp.float32),
            pltpu.SemaphoreType.DMA,
            pltpu.SemaphoreType.DMA,
            pltpu.SemaphoreType.DMA,
            pltpu.SemaphoreType.DMA,
            pltpu.SemaphoreType.DMA,
            pltpu.SemaphoreType.DMA,
            pltpu.SemaphoreType.DMA,
            pltpu.SemaphoreType.DMA,
            pltpu.SemaphoreType.DMA,
        ],
    )
    acc, _, _ = f(xt, epk)
    return acc


def kernel(user_embeds, item_embeds, edge_index, edge_weight):
    x = jnp.concatenate(
        [user_embeds, item_embeds,
         jnp.zeros((NP - N, D), jnp.float32)], axis=0)
    xt = jnp.transpose(x.reshape(NP, 2, H), (1, 0, 2))
    pad = EPAD - E
    zi = jnp.zeros((pad,), jnp.int32)
    epk = jnp.stack([
        jnp.concatenate([edge_index[0], zi]).reshape(ROWS, GROUP),
        jnp.concatenate([edge_index[1], zi]).reshape(ROWS, GROUP),
        jnp.concatenate(
            [lax.bitcast_convert_type(edge_weight, jnp.int32),
             zi]).reshape(ROWS, GROUP),
    ], axis=1)
    acc = _propagate(xt, epk)
    out = jnp.transpose(acc[:, :N], (1, 0, 2)).reshape(N, D)
    return (out[:N_USER], out[N_USER:])
